# baseline (speedup 1.0000x reference)
import functools

import jax
import jax.numpy as jnp
from jax import lax
from jax.experimental import pallas as pl
from jax.experimental.pallas import tpu as pltpu
from jax.experimental.pallas import tpu_sc as plsc

N = 10000
E = 320000
D = 128
D_OUT = 64

NC = 2
NS = 16
NW = NC * NS
EPT = E // NW
CHUNK = 32
KG = 4
EPTP = 10240
NCHP = EPTP // CHUNK
NGRP = NCHP // KG
PADR = 64
RPT = 624
RREM = N - NS * RPT


def _agg_body(x_hbm, src_hbm, dst_hbm, zeros_hbm, out_hbm,
              src_v, dst_v, rows_v, acc_sh, ixsems, gsems):
    c = lax.axis_index("c")
    s = lax.axis_index("s")
    wid = c * NS + s

    def issue_idx_group(g, p):
        for k in range(KG):
            b = KG * p + k
            off = wid * EPTP + (g * KG + k) * CHUNK
            pltpu.async_copy(src_hbm.at[pl.ds(off, CHUNK)], src_v.at[b],
                             ixsems[p])
            pltpu.async_copy(dst_hbm.at[pl.ds(off, CHUNK)], dst_v.at[b],
                             ixsems[p])

    def wait_idx_group(p):
        for k in range(KG):
            b = KG * p + k
            pltpu.make_async_copy(src_hbm.at[pl.ds(0, CHUNK)], src_v.at[b],
                                  ixsems[p]).wait()
            pltpu.make_async_copy(dst_hbm.at[pl.ds(0, CHUNK)], dst_v.at[b],
                                  ixsems[p]).wait()

    def issue_gather_group(p):
        for k in range(KG):
            b = KG * p + k
            pltpu.async_copy(x_hbm.at[src_v.at[b]], rows_v.at[b], gsems[p])

    def drain_gather_group(p):
        for k in range(KG):
            b = KG * p + k
            pltpu.make_async_copy(x_hbm.at[src_v.at[b]], rows_v.at[b],
                                  gsems[p]).wait()

    def scatter_group(p):
        for k in range(KG):
            b = KG * p + k
            pltpu.sync_copy(rows_v.at[b], acc_sh.at[dst_v.at[b]], add=True)

    issue_idx_group(0, 0)
    issue_idx_group(1, 1)
    pltpu.sync_copy(zeros_hbm.at[pl.ds(s * RPT, RPT)],
                    acc_sh.at[pl.ds(s * RPT, RPT)])

    @pl.when(s == NS - 1)
    def _():
        pltpu.sync_copy(zeros_hbm.at[pl.ds(NS * RPT, RREM)],
                        acc_sh.at[pl.ds(NS * RPT, RREM)])

    plsc.subcore_barrier()

    wait_idx_group(0)
    issue_gather_group(0)

    def step(g, p, do_idx, do_gather):
        drain_gather_group(p)
        if do_gather:
            wait_idx_group(1 - p)
            issue_gather_group(1 - p)
        scatter_group(p)
        if do_idx:
            issue_idx_group(g + 2, p)

    @pl.loop(0, NGRP - 2, step=2)
    def _(go):
        for q in range(2):
            step(go + q, q, True, True)

    for g in range(NGRP - 2, NGRP):
        step(g, g % 2, False, g + 1 < NGRP)

    plsc.subcore_barrier()
    pltpu.sync_copy(acc_sh.at[pl.ds(s * RPT, RPT)],
                    out_hbm.at[c, pl.ds(s * RPT, RPT)])

    @pl.when(s == NS - 1)
    def _():
        pltpu.sync_copy(acc_sh.at[pl.ds(NS * RPT, RREM)],
                        out_hbm.at[c, pl.ds(NS * RPT, RREM)])


@functools.partial(
    pl.kernel,
    out_type=jax.ShapeDtypeStruct((NC, N, D), jnp.float32),
    mesh=plsc.VectorSubcoreMesh(core_axis_name="c", subcore_axis_name="s",
                                num_cores=NC, num_subcores=NS),
    scratch_types=[
        pltpu.VMEM((2 * KG, CHUNK), jnp.int32),
        pltpu.VMEM((2 * KG, CHUNK), jnp.int32),
        pltpu.VMEM((2 * KG, CHUNK, D), jnp.float32),
        pltpu.VMEM_SHARED((N + PADR, D), jnp.float32),
        (pltpu.SemaphoreType.DMA, pltpu.SemaphoreType.DMA),
        (pltpu.SemaphoreType.DMA, pltpu.SemaphoreType.DMA),
    ],
    name="gin_sc_aggregate",
)
def _sc_aggregate(x_hbm, src_hbm, dst_hbm, zeros_hbm, out_hbm,
                  src_v, dst_v, rows_v, acc_sh, ixsems, gsems):
    _agg_body(x_hbm, src_hbm, dst_hbm, zeros_hbm, out_hbm,
              src_v, dst_v, rows_v, acc_sh, ixsems, gsems)


def _bn(u, g, b):
    mu = jnp.mean(u, axis=0, keepdims=True)
    var = jnp.mean((u - mu) * (u - mu), axis=0, keepdims=True)
    return (u - mu) * lax.rsqrt(var + 1e-5) * g + b


def _mlp_mid_body(x_ref, a0_ref, a1_ref, Wa_ref, ba_ref, ga_ref, bea_ref,
                  Wb_ref, bb_ref, gb_ref, beb_ref, out_ref):
    h = x_ref[...] + a0_ref[...] + a1_ref[...]
    u = jax.nn.relu(jnp.dot(h, Wa_ref[...],
                            preferred_element_type=jnp.float32) + ba_ref[...])
    u = _bn(u, ga_ref[...], bea_ref[...])
    v = jax.nn.relu(jnp.dot(u, Wb_ref[...],
                            preferred_element_type=jnp.float32) + bb_ref[...])
    v = _bn(v, gb_ref[...], beb_ref[...])
    out_ref[...] = jax.nn.relu(v)


def _mlp_fin_body(x_ref, a0_ref, a1_ref, Wa_ref, ba_ref, ga_ref, bea_ref,
                  Wb_ref, bb_ref, gb_ref, beb_ref, Wc_ref, bc_ref, out_ref):
    h = x_ref[...] + a0_ref[...] + a1_ref[...]
    u = jax.nn.relu(jnp.dot(h, Wa_ref[...],
                            preferred_element_type=jnp.float32) + ba_ref[...])
    u = _bn(u, ga_ref[...], bea_ref[...])
    v = jax.nn.relu(jnp.dot(u, Wb_ref[...],
                            preferred_element_type=jnp.float32) + bb_ref[...])
    v = _bn(v, gb_ref[...], beb_ref[...])
    v = jax.nn.relu(v)
    out_ref[...] = jnp.dot(v, Wc_ref[...],
                           preferred_element_type=jnp.float32) + bc_ref[...]


_mlp_mid = pl.pallas_call(
    _mlp_mid_body,
    out_shape=jax.ShapeDtypeStruct((N, D), jnp.float32),
)

_mlp_fin = pl.pallas_call(
    _mlp_fin_body,
    out_shape=jax.ShapeDtypeStruct((N, D_OUT), jnp.float32),
)


def kernel(x, edge_attr, edge_index, W0a, b0a, g0a, be0a, W0b, b0b, g0b, be0b,
           W1a, b1a, g1a, be1a, W1b, b1b, g1b, be1b, Wc, bc):
    pp = EPTP - EPT
    lanes = jnp.arange(pp, dtype=jnp.int32)[None, :]
    tiles = jnp.arange(NW, dtype=jnp.int32)[:, None]
    pad_src = (lanes + tiles * 313) % N
    pad_dst = N + (lanes + tiles * 7) % PADR
    src = jnp.concatenate(
        [edge_index[0].reshape(NW, EPT), pad_src], axis=1).reshape(-1)
    dst = jnp.concatenate(
        [edge_index[1].reshape(NW, EPT), pad_dst], axis=1).reshape(-1)
    zeros = jnp.zeros((N, D), jnp.float32)

    r2 = lambda b: b.reshape(1, -1)

    agg0 = _sc_aggregate(x, src, dst, zeros)
    h0 = _mlp_mid(x, agg0[0], agg0[1], W0a, r2(b0a), r2(g0a), r2(be0a),
                  W0b, r2(b0b), r2(g0b), r2(be0b))
    agg1 = _sc_aggregate(h0, src, dst, zeros)
    out = _mlp_fin(h0, agg1[0], agg1[1], W1a, r2(b1a), r2(g1a), r2(be1a),
                   W1b, r2(b1b), r2(g1b), r2(be1b), Wc, r2(bc))
    return out


# speedup vs baseline: 1.0477x; 1.0477x over previous
import functools

import jax
import jax.numpy as jnp
from jax import lax
from jax.experimental import pallas as pl
from jax.experimental.pallas import tpu as pltpu
from jax.experimental.pallas import tpu_sc as plsc

N = 10000
E = 320000
D = 128
D_OUT = 64

NC = 2
NS = 16
NW = NC * NS
EPT = E // NW
CHUNK = 48
KG = 3
EPTP = 10080
NCHP = EPTP // CHUNK
NGRP = NCHP // KG
PADR = 64
RPT = 624
RREM = N - NS * RPT


def _agg_body(x_hbm, src_hbm, dst_hbm, zeros_hbm, out_hbm,
              src_v, dst_v, rows_v, acc_sh, ixsems, gsems):
    c = lax.axis_index("c")
    s = lax.axis_index("s")
    wid = c * NS + s

    def issue_idx_group(g, p):
        for k in range(KG):
            b = KG * p + k
            off = wid * EPTP + (g * KG + k) * CHUNK
            pltpu.async_copy(src_hbm.at[pl.ds(off, CHUNK)], src_v.at[b],
                             ixsems[p])
            pltpu.async_copy(dst_hbm.at[pl.ds(off, CHUNK)], dst_v.at[b],
                             ixsems[p])

    def wait_idx_group(p):
        for k in range(KG):
            b = KG * p + k
            pltpu.make_async_copy(src_hbm.at[pl.ds(0, CHUNK)], src_v.at[b],
                                  ixsems[p]).wait()
            pltpu.make_async_copy(dst_hbm.at[pl.ds(0, CHUNK)], dst_v.at[b],
                                  ixsems[p]).wait()

    def issue_gather_group(p):
        for k in range(KG):
            b = KG * p + k
            pltpu.async_copy(x_hbm.at[src_v.at[b]], rows_v.at[b], gsems[p])

    def drain_gather_group(p):
        for k in range(KG):
            b = KG * p + k
            pltpu.make_async_copy(x_hbm.at[src_v.at[b]], rows_v.at[b],
                                  gsems[p]).wait()

    def scatter_group(p):
        for k in range(KG):
            b = KG * p + k
            pltpu.sync_copy(rows_v.at[b], acc_sh.at[dst_v.at[b]], add=True)

    issue_idx_group(0, 0)
    issue_idx_group(1, 1)
    pltpu.sync_copy(zeros_hbm.at[pl.ds(s * RPT, RPT)],
                    acc_sh.at[pl.ds(s * RPT, RPT)])

    @pl.when(s == NS - 1)
    def _():
        pltpu.sync_copy(zeros_hbm.at[pl.ds(NS * RPT, RREM)],
                        acc_sh.at[pl.ds(NS * RPT, RREM)])

    plsc.subcore_barrier()

    wait_idx_group(0)
    issue_gather_group(0)

    def step(g, p, do_idx, do_gather):
        drain_gather_group(p)
        if do_gather:
            wait_idx_group(1 - p)
            issue_gather_group(1 - p)
        scatter_group(p)
        if do_idx:
            issue_idx_group(g + 2, p)

    @pl.loop(0, NGRP - 2, step=2)
    def _(go):
        for q in range(2):
            step(go + q, q, True, True)

    for g in range(NGRP - 2, NGRP):
        step(g, g % 2, False, g + 1 < NGRP)

    plsc.subcore_barrier()
    pltpu.sync_copy(acc_sh.at[pl.ds(s * RPT, RPT)],
                    out_hbm.at[c, pl.ds(s * RPT, RPT)])

    @pl.when(s == NS - 1)
    def _():
        pltpu.sync_copy(acc_sh.at[pl.ds(NS * RPT, RREM)],
                        out_hbm.at[c, pl.ds(NS * RPT, RREM)])


@functools.partial(
    pl.kernel,
    out_type=jax.ShapeDtypeStruct((NC, N, D), jnp.float32),
    mesh=plsc.VectorSubcoreMesh(core_axis_name="c", subcore_axis_name="s",
                                num_cores=NC, num_subcores=NS),
    scratch_types=[
        pltpu.VMEM((2 * KG, CHUNK), jnp.int32),
        pltpu.VMEM((2 * KG, CHUNK), jnp.int32),
        pltpu.VMEM((2 * KG, CHUNK, D), jnp.float32),
        pltpu.VMEM_SHARED((N + PADR, D), jnp.float32),
        (pltpu.SemaphoreType.DMA, pltpu.SemaphoreType.DMA),
        (pltpu.SemaphoreType.DMA, pltpu.SemaphoreType.DMA),
    ],
    name="gin_sc_aggregate",
)
def _sc_aggregate(x_hbm, src_hbm, dst_hbm, zeros_hbm, out_hbm,
                  src_v, dst_v, rows_v, acc_sh, ixsems, gsems):
    _agg_body(x_hbm, src_hbm, dst_hbm, zeros_hbm, out_hbm,
              src_v, dst_v, rows_v, acc_sh, ixsems, gsems)


def _bn(u, g, b):
    mu = jnp.mean(u, axis=0, keepdims=True)
    var = jnp.mean((u - mu) * (u - mu), axis=0, keepdims=True)
    return (u - mu) * lax.rsqrt(var + 1e-5) * g + b


def _mlp_mid_body(x_ref, a0_ref, a1_ref, Wa_ref, ba_ref, ga_ref, bea_ref,
                  Wb_ref, bb_ref, gb_ref, beb_ref, out_ref):
    h = x_ref[...] + a0_ref[...] + a1_ref[...]
    u = jax.nn.relu(jnp.dot(h, Wa_ref[...],
                            preferred_element_type=jnp.float32) + ba_ref[...])
    u = _bn(u, ga_ref[...], bea_ref[...])
    v = jax.nn.relu(jnp.dot(u, Wb_ref[...],
                            preferred_element_type=jnp.float32) + bb_ref[...])
    v = _bn(v, gb_ref[...], beb_ref[...])
    out_ref[...] = jax.nn.relu(v)


def _mlp_fin_body(x_ref, a0_ref, a1_ref, Wa_ref, ba_ref, ga_ref, bea_ref,
                  Wb_ref, bb_ref, gb_ref, beb_ref, Wc_ref, bc_ref, out_ref):
    h = x_ref[...] + a0_ref[...] + a1_ref[...]
    u = jax.nn.relu(jnp.dot(h, Wa_ref[...],
                            preferred_element_type=jnp.float32) + ba_ref[...])
    u = _bn(u, ga_ref[...], bea_ref[...])
    v = jax.nn.relu(jnp.dot(u, Wb_ref[...],
                            preferred_element_type=jnp.float32) + bb_ref[...])
    v = _bn(v, gb_ref[...], beb_ref[...])
    v = jax.nn.relu(v)
    out_ref[...] = jnp.dot(v, Wc_ref[...],
                           preferred_element_type=jnp.float32) + bc_ref[...]


_mlp_mid = pl.pallas_call(
    _mlp_mid_body,
    out_shape=jax.ShapeDtypeStruct((N, D), jnp.float32),
)

_mlp_fin = pl.pallas_call(
    _mlp_fin_body,
    out_shape=jax.ShapeDtypeStruct((N, D_OUT), jnp.float32),
)


def kernel(x, edge_attr, edge_index, W0a, b0a, g0a, be0a, W0b, b0b, g0b, be0b,
           W1a, b1a, g1a, be1a, W1b, b1b, g1b, be1b, Wc, bc):
    pp = EPTP - EPT
    lanes = jnp.arange(pp, dtype=jnp.int32)[None, :]
    tiles = jnp.arange(NW, dtype=jnp.int32)[:, None]
    pad_src = (lanes + tiles * 313) % N
    pad_dst = N + (lanes + tiles * 7) % PADR
    src = jnp.concatenate(
        [edge_index[0].reshape(NW, EPT), pad_src], axis=1).reshape(-1)
    dst = jnp.concatenate(
        [edge_index[1].reshape(NW, EPT), pad_dst], axis=1).reshape(-1)
    zeros = jnp.zeros((N, D), jnp.float32)

    r2 = lambda b: b.reshape(1, -1)

    agg0 = _sc_aggregate(x, src, dst, zeros)
    h0 = _mlp_mid(x, agg0[0], agg0[1], W0a, r2(b0a), r2(g0a), r2(be0a),
                  W0b, r2(b0b), r2(g0b), r2(be0b))
    agg1 = _sc_aggregate(h0, src, dst, zeros)
    out = _mlp_fin(h0, agg1[0], agg1[1], W1a, r2(b1a), r2(g1a), r2(be1a),
                   W1b, r2(b1b), r2(g1b), r2(be1b), Wc, r2(bc))
    return out


# speedup vs baseline: 1.0903x; 1.0407x over previous
import functools

import jax
import jax.numpy as jnp
from jax import lax
from jax.experimental import pallas as pl
from jax.experimental.pallas import tpu as pltpu
from jax.experimental.pallas import tpu_sc as plsc

N = 10000
E = 320000
D = 128
D_OUT = 64

NC = 2
NS = 16
NW = NC * NS
EPT = E // NW
CHUNK = 48
KG = 3
EPTP = 10080
NCHP = EPTP // CHUNK
NGRP = NCHP // KG
PADR = 64
RPT = 624
RREM = N - NS * RPT


def _agg_body(x_hbm, src_hbm, dst_hbm, zeros_hbm, out_hbm,
              src_v, dst_v, rows_v, acc_sh, ixsems, gsems):
    c = lax.axis_index("c")
    s = lax.axis_index("s")
    wid = c * NS + s

    def issue_idx_group(g, p):
        for k in range(KG):
            b = KG * p + k
            off = wid * EPTP + (g * KG + k) * CHUNK
            pltpu.async_copy(src_hbm.at[pl.ds(off, CHUNK)], src_v.at[b],
                             ixsems[p])
            pltpu.async_copy(dst_hbm.at[pl.ds(off, CHUNK)], dst_v.at[b],
                             ixsems[p])

    def wait_idx_group(p):
        for k in range(KG):
            b = KG * p + k
            pltpu.make_async_copy(src_hbm.at[pl.ds(0, CHUNK)], src_v.at[b],
                                  ixsems[p]).wait()
            pltpu.make_async_copy(dst_hbm.at[pl.ds(0, CHUNK)], dst_v.at[b],
                                  ixsems[p]).wait()

    def issue_gather_group(p):
        for k in range(KG):
            b = KG * p + k
            pltpu.async_copy(x_hbm.at[src_v.at[b]], rows_v.at[b], gsems[p])

    def drain_gather_group(p):
        for k in range(KG):
            b = KG * p + k
            pltpu.make_async_copy(x_hbm.at[src_v.at[b]], rows_v.at[b],
                                  gsems[p]).wait()

    def scatter_group(p):
        for k in range(KG):
            b = KG * p + k
            pltpu.sync_copy(rows_v.at[b], acc_sh.at[dst_v.at[b]], add=True)

    issue_idx_group(0, 0)
    issue_idx_group(1, 1)
    pltpu.sync_copy(zeros_hbm.at[pl.ds(s * RPT, RPT)],
                    acc_sh.at[pl.ds(s * RPT, RPT)])

    @pl.when(s == NS - 1)
    def _():
        pltpu.sync_copy(zeros_hbm.at[pl.ds(NS * RPT, RREM)],
                        acc_sh.at[pl.ds(NS * RPT, RREM)])

    plsc.subcore_barrier()

    wait_idx_group(0)
    issue_gather_group(0)

    def step(g, p, do_idx, do_gather):
        drain_gather_group(p)
        if do_gather:
            wait_idx_group(1 - p)
            issue_gather_group(1 - p)
        scatter_group(p)
        if do_idx:
            issue_idx_group(g + 2, p)

    @pl.loop(0, NGRP - 2, step=2)
    def _(go):
        for q in range(2):
            step(go + q, q, True, True)

    for g in range(NGRP - 2, NGRP):
        step(g, g % 2, False, g + 1 < NGRP)

    plsc.subcore_barrier()
    pltpu.sync_copy(acc_sh.at[pl.ds(s * RPT, RPT)],
                    out_hbm.at[c, pl.ds(s * RPT, RPT)])

    @pl.when(s == NS - 1)
    def _():
        pltpu.sync_copy(acc_sh.at[pl.ds(NS * RPT, RREM)],
                        out_hbm.at[c, pl.ds(NS * RPT, RREM)])


@functools.partial(
    pl.kernel,
    out_type=jax.ShapeDtypeStruct((NC, N, D), jnp.float32),
    mesh=plsc.VectorSubcoreMesh(core_axis_name="c", subcore_axis_name="s",
                                num_cores=NC, num_subcores=NS),
    scratch_types=[
        pltpu.VMEM((2 * KG, CHUNK), jnp.int32),
        pltpu.VMEM((2 * KG, CHUNK), jnp.int32),
        pltpu.VMEM((2 * KG, CHUNK, D), jnp.float32),
        pltpu.VMEM_SHARED((N + PADR, D), jnp.float32),
        (pltpu.SemaphoreType.DMA, pltpu.SemaphoreType.DMA),
        (pltpu.SemaphoreType.DMA, pltpu.SemaphoreType.DMA),
    ],
    name="gin_sc_aggregate",
)
def _sc_aggregate(x_hbm, src_hbm, dst_hbm, zeros_hbm, out_hbm,
                  src_v, dst_v, rows_v, acc_sh, ixsems, gsems):
    _agg_body(x_hbm, src_hbm, dst_hbm, zeros_hbm, out_hbm,
              src_v, dst_v, rows_v, acc_sh, ixsems, gsems)


def _bn(u, g, b):
    mu = jnp.mean(u, axis=0, keepdims=True)
    var = jnp.mean((u - mu) * (u - mu), axis=0, keepdims=True)
    return (u - mu) * lax.rsqrt(var + 1e-5) * g + b


def _mlp_mid_body(x_ref, agg_ref, Wa_ref, ba_ref, ga_ref, bea_ref,
                  Wb_ref, bb_ref, gb_ref, beb_ref, out_ref):
    h = x_ref[...] + agg_ref[0] + agg_ref[1]
    u = jax.nn.relu(jnp.dot(h, Wa_ref[...],
                            preferred_element_type=jnp.float32) + ba_ref[...])
    u = _bn(u, ga_ref[...], bea_ref[...])
    v = jax.nn.relu(jnp.dot(u, Wb_ref[...],
                            preferred_element_type=jnp.float32) + bb_ref[...])
    v = _bn(v, gb_ref[...], beb_ref[...])
    out_ref[...] = jax.nn.relu(v)


def _mlp_fin_body(x_ref, agg_ref, Wa_ref, ba_ref, ga_ref, bea_ref,
                  Wb_ref, bb_ref, gb_ref, beb_ref, Wc_ref, bc_ref, out_ref):
    h = x_ref[...] + agg_ref[0] + agg_ref[1]
    u = jax.nn.relu(jnp.dot(h, Wa_ref[...],
                            preferred_element_type=jnp.float32) + ba_ref[...])
    u = _bn(u, ga_ref[...], bea_ref[...])
    v = jax.nn.relu(jnp.dot(u, Wb_ref[...],
                            preferred_element_type=jnp.float32) + bb_ref[...])
    v = _bn(v, gb_ref[...], beb_ref[...])
    v = jax.nn.relu(v)
    out_ref[...] = jnp.dot(v, Wc_ref[...],
                           preferred_element_type=jnp.float32) + bc_ref[...]


_mlp_mid = pl.pallas_call(
    _mlp_mid_body,
    out_shape=jax.ShapeDtypeStruct((N, D), jnp.float32),
)

_mlp_fin = pl.pallas_call(
    _mlp_fin_body,
    out_shape=jax.ShapeDtypeStruct((N, D_OUT), jnp.float32),
)


def kernel(x, edge_attr, edge_index, W0a, b0a, g0a, be0a, W0b, b0b, g0b, be0b,
           W1a, b1a, g1a, be1a, W1b, b1b, g1b, be1b, Wc, bc):
    pp = EPTP - EPT
    lanes = jnp.arange(pp, dtype=jnp.int32)[None, :]
    tiles = jnp.arange(NW, dtype=jnp.int32)[:, None]
    pad_src = (lanes + tiles * 313) % N
    pad_dst = N + (lanes + tiles * 7) % PADR
    src = jnp.concatenate(
        [edge_index[0].reshape(NW, EPT), pad_src], axis=1).reshape(-1)
    dst = jnp.concatenate(
        [edge_index[1].reshape(NW, EPT), pad_dst], axis=1).reshape(-1)
    zeros = jnp.zeros((N, D), jnp.float32)

    r2 = lambda b: b.reshape(1, -1)

    agg0 = _sc_aggregate(x, src, dst, zeros)
    h0 = _mlp_mid(x, agg0, W0a, r2(b0a), r2(g0a), r2(be0a),
                  W0b, r2(b0b), r2(g0b), r2(be0b))
    agg1 = _sc_aggregate(h0, src, dst, zeros)
    out = _mlp_fin(h0, agg1, W1a, r2(b1a), r2(g1a), r2(be1a),
                   W1b, r2(b1b), r2(g1b), r2(be1b), Wc, r2(bc))
    return out


# speedup vs baseline: 1.1358x; 1.0417x over previous
import functools

import jax
import jax.numpy as jnp
from jax import lax
from jax.experimental import pallas as pl
from jax.experimental.pallas import tpu as pltpu
from jax.experimental.pallas import tpu_sc as plsc

N = 10000
E = 320000
D = 128
D_OUT = 64

NC = 2
NS = 16
NW = NC * NS
EPT = E // NW
CHUNK = 48
KG = 3
NGRP = 69
TCH0 = NGRP * KG
TAIL = EPT - (TCH0 + 1) * CHUNK
PADR = 64
RPT = 624
RREM = N - NS * RPT


def _agg_body(x_hbm, er_hbm, zeros_hbm, out_hbm,
              src_v, dst_v, rows_v, acc_sh, ixsems, gsems):
    c = lax.axis_index("c")
    s = lax.axis_index("s")
    wid = c * NS + s
    sbase = wid * EPT
    dbase = E + wid * EPT

    def issue_idx_chunk(ch, b, p, n=CHUNK):
        pltpu.async_copy(er_hbm.at[pl.ds(sbase + ch * CHUNK, n)],
                         src_v.at[b, pl.ds(0, n)], ixsems[p])
        pltpu.async_copy(er_hbm.at[pl.ds(dbase + ch * CHUNK, n)],
                         dst_v.at[b, pl.ds(0, n)], ixsems[p])

    def wait_idx_chunk(b, p, n=CHUNK):
        pltpu.make_async_copy(er_hbm.at[pl.ds(0, n)],
                              src_v.at[b, pl.ds(0, n)], ixsems[p]).wait()
        pltpu.make_async_copy(er_hbm.at[pl.ds(0, n)],
                              dst_v.at[b, pl.ds(0, n)], ixsems[p]).wait()

    def issue_idx_group(g, p):
        for k in range(KG):
            issue_idx_chunk(g * KG + k, KG * p + k, p)

    def wait_idx_group(p):
        for k in range(KG):
            wait_idx_chunk(KG * p + k, p)

    def issue_gather_group(p):
        for k in range(KG):
            b = KG * p + k
            pltpu.async_copy(x_hbm.at[src_v.at[b]], rows_v.at[b], gsems[p])

    def drain_gather_group(p):
        for k in range(KG):
            b = KG * p + k
            pltpu.make_async_copy(x_hbm.at[src_v.at[b]], rows_v.at[b],
                                  gsems[p]).wait()

    def scatter_group(p):
        for k in range(KG):
            b = KG * p + k
            pltpu.sync_copy(rows_v.at[b], acc_sh.at[dst_v.at[b]], add=True)

    issue_idx_group(0, 0)
    issue_idx_group(1, 1)
    pltpu.sync_copy(zeros_hbm.at[pl.ds(s * RPT, RPT)],
                    acc_sh.at[pl.ds(s * RPT, RPT)])

    @pl.when(s == NS - 1)
    def _():
        pltpu.sync_copy(zeros_hbm.at[pl.ds(NS * RPT, RREM)],
                        acc_sh.at[pl.ds(NS * RPT, RREM)])

    plsc.subcore_barrier()

    wait_idx_group(0)
    issue_gather_group(0)

    def step(g, p, do_idx, do_gather):
        drain_gather_group(p)
        if do_gather:
            wait_idx_group(1 - p)
            issue_gather_group(1 - p)
        scatter_group(p)
        if do_idx:
            issue_idx_group(g + 2, p)

    @pl.loop(0, NGRP - 3, step=2)
    def _(go):
        for q in range(2):
            step(go + q, q, True, True)

    def issue_idx_ragged():
        issue_idx_chunk(TCH0, KG, 1)
        issue_idx_chunk(TCH0 + 1, KG + 1, 1, n=TAIL)
        iota = lax.iota(jnp.int32, 16)
        for k in range((CHUNK - TAIL) // 16):
            pad = N + lax.rem(iota + s * 16 + k * 16, PADR)
            dst_v[KG + 1, pl.ds(TAIL + k * 16, 16)] = pad

    def wait_idx_ragged():
        wait_idx_chunk(KG, 1)
        wait_idx_chunk(KG + 1, 1, n=TAIL)

    def issue_gather_ragged():
        for b in (KG, KG + 1):
            pltpu.async_copy(x_hbm.at[src_v.at[b]], rows_v.at[b], gsems[1])

    def drain_scatter_ragged():
        for b in (KG, KG + 1):
            pltpu.make_async_copy(x_hbm.at[src_v.at[b]], rows_v.at[b],
                                  gsems[1]).wait()
        for b in (KG, KG + 1):
            pltpu.sync_copy(rows_v.at[b], acc_sh.at[dst_v.at[b]], add=True)

    g0 = NGRP - 3
    drain_gather_group(0)
    wait_idx_group(1)
    issue_gather_group(1)
    scatter_group(0)
    issue_idx_group(NGRP - 1, 0)

    drain_gather_group(1)
    wait_idx_group(0)
    issue_gather_group(0)
    scatter_group(1)
    issue_idx_ragged()

    drain_gather_group(0)
    wait_idx_ragged()
    issue_gather_ragged()
    scatter_group(0)

    drain_scatter_ragged()

    plsc.subcore_barrier()
    pltpu.sync_copy(acc_sh.at[pl.ds(s * RPT, RPT)],
                    out_hbm.at[c, pl.ds(s * RPT, RPT)])

    @pl.when(s == NS - 1)
    def _():
        pltpu.sync_copy(acc_sh.at[pl.ds(NS * RPT, RREM)],
                        out_hbm.at[c, pl.ds(NS * RPT, RREM)])


@functools.partial(
    pl.kernel,
    out_type=jax.ShapeDtypeStruct((NC, N, D), jnp.float32),
    mesh=plsc.VectorSubcoreMesh(core_axis_name="c", subcore_axis_name="s",
                                num_cores=NC, num_subcores=NS),
    scratch_types=[
        pltpu.VMEM((2 * KG, CHUNK), jnp.int32),
        pltpu.VMEM((2 * KG, CHUNK), jnp.int32),
        pltpu.VMEM((2 * KG, CHUNK, D), jnp.float32),
        pltpu.VMEM_SHARED((N + PADR, D), jnp.float32),
        (pltpu.SemaphoreType.DMA, pltpu.SemaphoreType.DMA),
        (pltpu.SemaphoreType.DMA, pltpu.SemaphoreType.DMA),
    ],
    name="gin_sc_aggregate",
)
def _sc_aggregate(x_hbm, er_hbm, zeros_hbm, out_hbm,
                  src_v, dst_v, rows_v, acc_sh, ixsems, gsems):
    _agg_body(x_hbm, er_hbm, zeros_hbm, out_hbm,
              src_v, dst_v, rows_v, acc_sh, ixsems, gsems)


def _bn(u, g, b):
    mu = jnp.mean(u, axis=0, keepdims=True)
    var = jnp.mean((u - mu) * (u - mu), axis=0, keepdims=True)
    return (u - mu) * lax.rsqrt(var + 1e-5) * g + b


def _mlp_mid_body(x_ref, agg_ref, Wa_ref, ba_ref, ga_ref, bea_ref,
                  Wb_ref, bb_ref, gb_ref, beb_ref, out_ref):
    h = x_ref[...] + agg_ref[0] + agg_ref[1]
    u = jax.nn.relu(jnp.dot(h, Wa_ref[...],
                            preferred_element_type=jnp.float32) + ba_ref[...])
    u = _bn(u, ga_ref[...], bea_ref[...])
    v = jax.nn.relu(jnp.dot(u, Wb_ref[...],
                            preferred_element_type=jnp.float32) + bb_ref[...])
    v = _bn(v, gb_ref[...], beb_ref[...])
    out_ref[...] = jax.nn.relu(v)


def _mlp_fin_body(x_ref, agg_ref, Wa_ref, ba_ref, ga_ref, bea_ref,
                  Wb_ref, bb_ref, gb_ref, beb_ref, Wc_ref, bc_ref, out_ref):
    h = x_ref[...] + agg_ref[0] + agg_ref[1]
    u = jax.nn.relu(jnp.dot(h, Wa_ref[...],
                            preferred_element_type=jnp.float32) + ba_ref[...])
    u = _bn(u, ga_ref[...], bea_ref[...])
    v = jax.nn.relu(jnp.dot(u, Wb_ref[...],
                            preferred_element_type=jnp.float32) + bb_ref[...])
    v = _bn(v, gb_ref[...], beb_ref[...])
    v = jax.nn.relu(v)
    out_ref[...] = jnp.dot(v, Wc_ref[...],
                           preferred_element_type=jnp.float32) + bc_ref[...]


_mlp_mid = pl.pallas_call(
    _mlp_mid_body,
    out_shape=jax.ShapeDtypeStruct((N, D), jnp.float32),
)

_mlp_fin = pl.pallas_call(
    _mlp_fin_body,
    out_shape=jax.ShapeDtypeStruct((N, D_OUT), jnp.float32),
)


def kernel(x, edge_attr, edge_index, W0a, b0a, g0a, be0a, W0b, b0b, g0b, be0b,
           W1a, b1a, g1a, be1a, W1b, b1b, g1b, be1b, Wc, bc):
    er = edge_index.reshape(-1)
    zeros = jnp.zeros((N, D), jnp.float32)

    r2 = lambda b: b.reshape(1, -1)

    agg0 = _sc_aggregate(x, er, zeros)
    h0 = _mlp_mid(x, agg0, W0a, r2(b0a), r2(g0a), r2(be0a),
                  W0b, r2(b0b), r2(g0b), r2(be0b))
    agg1 = _sc_aggregate(h0, er, zeros)
    out = _mlp_fin(h0, agg1, W1a, r2(b1a), r2(g1a), r2(be1a),
                   W1b, r2(b1b), r2(g1b), r2(be1b), Wc, r2(bc))
    return out


# speedup vs baseline: 1.1391x; 1.0029x over previous
import functools

import jax
import jax.numpy as jnp
from jax import lax
from jax.experimental import pallas as pl
from jax.experimental.pallas import tpu as pltpu
from jax.experimental.pallas import tpu_sc as plsc

N = 10000
E = 320000
D = 128
D_OUT = 64

NC = 2
NS = 16
NW = NC * NS
EPT = E // NW
CHUNK = 48
KG = 3
NGRP = 69
TCH0 = NGRP * KG
TAIL = EPT - (TCH0 + 1) * CHUNK
PADR = 64
RPT = 624
RREM = N - NS * RPT


def _agg_body(x_hbm, er_hbm, zeros_hbm, out_hbm,
              src_v, dst_v, rows_v, acc_sh, ixsems, gsems):
    c = lax.axis_index("c")
    s = lax.axis_index("s")
    wid = c * NS + s
    sbase = wid * EPT
    dbase = E + wid * EPT

    def issue_idx_chunk(ch, b, p, n=CHUNK):
        pltpu.async_copy(er_hbm.at[pl.ds(sbase + ch * CHUNK, n)],
                         src_v.at[b, pl.ds(0, n)], ixsems[p])
        pltpu.async_copy(er_hbm.at[pl.ds(dbase + ch * CHUNK, n)],
                         dst_v.at[b, pl.ds(0, n)], ixsems[p])

    def wait_idx_chunk(b, p, n=CHUNK):
        pltpu.make_async_copy(er_hbm.at[pl.ds(0, n)],
                              src_v.at[b, pl.ds(0, n)], ixsems[p]).wait()
        pltpu.make_async_copy(er_hbm.at[pl.ds(0, n)],
                              dst_v.at[b, pl.ds(0, n)], ixsems[p]).wait()

    def issue_idx_group(g, p):
        for k in range(KG):
            issue_idx_chunk(g * KG + k, KG * p + k, p)

    def wait_idx_group(p):
        for k in range(KG):
            wait_idx_chunk(KG * p + k, p)

    def issue_gather_group(p):
        for k in range(KG):
            b = KG * p + k
            pltpu.async_copy(x_hbm.at[src_v.at[b]], rows_v.at[b], gsems[p])

    def drain_gather_group(p):
        for k in range(KG):
            b = KG * p + k
            pltpu.make_async_copy(x_hbm.at[src_v.at[b]], rows_v.at[b],
                                  gsems[p]).wait()

    def scatter_group(p):
        for k in range(KG):
            b = KG * p + k
            pltpu.sync_copy(rows_v.at[b], acc_sh.at[dst_v.at[b]], add=True)

    issue_idx_group(0, 0)
    issue_idx_group(1, 1)
    zd = pltpu.async_copy(zeros_hbm, acc_sh.at[pl.ds(s * RPT, RPT)],
                          gsems[1])

    @pl.when(s == NS - 1)
    def _():
        pltpu.sync_copy(zeros_hbm.at[pl.ds(0, RREM)],
                        acc_sh.at[pl.ds(NS * RPT, RREM)])

    zd.wait()
    plsc.subcore_barrier()

    wait_idx_group(0)
    issue_gather_group(0)

    def step(g, p, do_idx, do_gather):
        drain_gather_group(p)
        if do_gather:
            wait_idx_group(1 - p)
            issue_gather_group(1 - p)
        scatter_group(p)
        if do_idx:
            issue_idx_group(g + 2, p)

    @pl.loop(0, NGRP - 3, step=2)
    def _(go):
        for q in range(2):
            step(go + q, q, True, True)

    def issue_idx_ragged():
        issue_idx_chunk(TCH0, KG, 1)
        issue_idx_chunk(TCH0 + 1, KG + 1, 1, n=TAIL)
        iota = lax.iota(jnp.int32, 16)
        for k in range((CHUNK - TAIL) // 16):
            pad = N + lax.rem(iota + s * 16 + k * 16, PADR)
            dst_v[KG + 1, pl.ds(TAIL + k * 16, 16)] = pad

    def wait_idx_ragged():
        wait_idx_chunk(KG, 1)
        wait_idx_chunk(KG + 1, 1, n=TAIL)

    def issue_gather_ragged():
        for b in (KG, KG + 1):
            pltpu.async_copy(x_hbm.at[src_v.at[b]], rows_v.at[b], gsems[1])

    def drain_scatter_ragged():
        for b in (KG, KG + 1):
            pltpu.make_async_copy(x_hbm.at[src_v.at[b]], rows_v.at[b],
                                  gsems[1]).wait()
        for b in (KG, KG + 1):
            pltpu.sync_copy(rows_v.at[b], acc_sh.at[dst_v.at[b]], add=True)

    g0 = NGRP - 3
    drain_gather_group(0)
    wait_idx_group(1)
    issue_gather_group(1)
    scatter_group(0)
    issue_idx_group(NGRP - 1, 0)

    drain_gather_group(1)
    wait_idx_group(0)
    issue_gather_group(0)
    scatter_group(1)
    issue_idx_ragged()

    drain_gather_group(0)
    wait_idx_ragged()
    issue_gather_ragged()
    scatter_group(0)

    drain_scatter_ragged()

    plsc.subcore_barrier()
    pltpu.sync_copy(acc_sh.at[pl.ds(s * RPT, RPT)],
                    out_hbm.at[c, pl.ds(s * RPT, RPT)])

    @pl.when(s == NS - 1)
    def _():
        pltpu.sync_copy(acc_sh.at[pl.ds(NS * RPT, RREM)],
                        out_hbm.at[c, pl.ds(NS * RPT, RREM)])


@functools.partial(
    pl.kernel,
    out_type=jax.ShapeDtypeStruct((NC, N, D), jnp.float32),
    mesh=plsc.VectorSubcoreMesh(core_axis_name="c", subcore_axis_name="s",
                                num_cores=NC, num_subcores=NS),
    scratch_types=[
        pltpu.VMEM((2 * KG, CHUNK), jnp.int32),
        pltpu.VMEM((2 * KG, CHUNK), jnp.int32),
        pltpu.VMEM((2 * KG, CHUNK, D), jnp.float32),
        pltpu.VMEM_SHARED((N + PADR, D), jnp.float32),
        (pltpu.SemaphoreType.DMA, pltpu.SemaphoreType.DMA),
        (pltpu.SemaphoreType.DMA, pltpu.SemaphoreType.DMA),
    ],
    name="gin_sc_aggregate",
)
def _sc_aggregate(x_hbm, er_hbm, zeros_hbm, out_hbm,
                  src_v, dst_v, rows_v, acc_sh, ixsems, gsems):
    _agg_body(x_hbm, er_hbm, zeros_hbm, out_hbm,
              src_v, dst_v, rows_v, acc_sh, ixsems, gsems)


def _bn(u, g, b):
    mu = jnp.mean(u, axis=0, keepdims=True)
    var = jnp.mean((u - mu) * (u - mu), axis=0, keepdims=True)
    return (u - mu) * lax.rsqrt(var + 1e-5) * g + b


def _mlp_mid_body(x_ref, agg_ref, Wa_ref, ba_ref, ga_ref, bea_ref,
                  Wb_ref, bb_ref, gb_ref, beb_ref, out_ref):
    h = x_ref[...] + agg_ref[0] + agg_ref[1]
    u = jax.nn.relu(jnp.dot(h, Wa_ref[...],
                            preferred_element_type=jnp.float32) + ba_ref[...])
    u = _bn(u, ga_ref[...], bea_ref[...])
    v = jax.nn.relu(jnp.dot(u, Wb_ref[...],
                            preferred_element_type=jnp.float32) + bb_ref[...])
    v = _bn(v, gb_ref[...], beb_ref[...])
    out_ref[...] = jax.nn.relu(v)


def _mlp_fin_body(x_ref, agg_ref, Wa_ref, ba_ref, ga_ref, bea_ref,
                  Wb_ref, bb_ref, gb_ref, beb_ref, Wc_ref, bc_ref, out_ref):
    h = x_ref[...] + agg_ref[0] + agg_ref[1]
    u = jax.nn.relu(jnp.dot(h, Wa_ref[...],
                            preferred_element_type=jnp.float32) + ba_ref[...])
    u = _bn(u, ga_ref[...], bea_ref[...])
    v = jax.nn.relu(jnp.dot(u, Wb_ref[...],
                            preferred_element_type=jnp.float32) + bb_ref[...])
    v = _bn(v, gb_ref[...], beb_ref[...])
    v = jax.nn.relu(v)
    out_ref[...] = jnp.dot(v, Wc_ref[...],
                           preferred_element_type=jnp.float32) + bc_ref[...]


_mlp_mid = pl.pallas_call(
    _mlp_mid_body,
    out_shape=jax.ShapeDtypeStruct((N, D), jnp.float32),
)

_mlp_fin = pl.pallas_call(
    _mlp_fin_body,
    out_shape=jax.ShapeDtypeStruct((N, D_OUT), jnp.float32),
)


def kernel(x, edge_attr, edge_index, W0a, b0a, g0a, be0a, W0b, b0b, g0b, be0b,
           W1a, b1a, g1a, be1a, W1b, b1b, g1b, be1b, Wc, bc):
    er = edge_index.reshape(-1)
    zeros = jnp.zeros((RPT, D), jnp.float32)

    r2 = lambda b: b.reshape(1, -1)

    agg0 = _sc_aggregate(x, er, zeros)
    h0 = _mlp_mid(x, agg0, W0a, r2(b0a), r2(g0a), r2(be0a),
                  W0b, r2(b0b), r2(g0b), r2(be0b))
    agg1 = _sc_aggregate(h0, er, zeros)
    out = _mlp_fin(h0, agg1, W1a, r2(b1a), r2(g1a), r2(be1a),
                   W1b, r2(b1b), r2(g1b), r2(be1b), Wc, r2(bc))
    return out
